# Initial kernel scaffold; baseline (speedup 1.0000x reference)
import jax
import jax.numpy as jnp
from jax.experimental import pallas as pl


def kernel(q, k, W_qs, b_qs, W_ks, b_ks):
    raise NotImplementedError("write your pallas kernel here")


# speedup vs baseline: 43.1830x; 43.1830x over previous
import functools
import math

import jax
import jax.numpy as jnp
from jax.experimental import pallas as pl
from jax.experimental.pallas import tpu as pltpu

TOPK = 5
EPS = 1e-8
KT = 2048


def _stats_kernel(q_ref, k_ref, wqt_ref, bq_ref, wkt_ref, bk_ref,
                  w_ref, idx_ref,
                  qp_ref, m_ref, z_ref, tv_ref, ti_ref,
                  *, n_k, kt, nsteps, scale):
    i = pl.program_id(0)
    nq = q_ref.shape[0]

    @pl.when(i == 0)
    def _init():
        qp_ref[...] = (jnp.dot(q_ref[...], wqt_ref[...],
                               preferred_element_type=jnp.float32)
                       + bq_ref[...])
        m_ref[...] = jnp.full((nq, 1), -jnp.inf, jnp.float32)
        z_ref[...] = jnp.zeros((nq, 1), jnp.float32)
        tv_ref[...] = jnp.full((nq, TOPK), -jnp.inf, jnp.float32)
        ti_ref[...] = jnp.zeros((nq, TOPK), jnp.int32)

    kp = (jnp.dot(k_ref[...], wkt_ref[...], preferred_element_type=jnp.float32)
          + bk_ref[...])
    s = jax.lax.dot_general(qp_ref[...], kp, (((1,), (1,)), ((), ())),
                            preferred_element_type=jnp.float32) * scale

    col = jax.lax.broadcasted_iota(jnp.int32, (1, kt), 1) + i * kt
    s = jnp.where(col < n_k, s, -jnp.inf)

    tmax = jnp.max(s, axis=1, keepdims=True)
    m_old = m_ref[...]
    m_new = jnp.maximum(m_old, tmax)
    z_ref[...] = (z_ref[...] * jnp.exp(m_old - m_new)
                  + jnp.sum(jnp.exp(s - m_new), axis=1, keepdims=True))
    m_ref[...] = m_new

    rv = [tv_ref[:, j:j + 1] for j in range(TOPK)]
    ri = [ti_ref[:, j:j + 1] for j in range(TOPK)]
    cur = s
    for _ in range(TOPK):
        v = jnp.max(cur, axis=1, keepdims=True)
        vi = jnp.min(jnp.where(cur == v, col, jnp.int32(2 ** 30)),
                     axis=1, keepdims=True)
        cur = jnp.where(col == vi, -jnp.inf, cur)
        new_rv, new_ri = [], []
        prev_gt = None
        for j in range(TOPK):
            gt = v > rv[j]
            if j == 0:
                nv = jnp.where(gt, v, rv[j])
                ni = jnp.where(gt, vi, ri[j])
            else:
                inc_v = jnp.where(prev_gt, rv[j - 1], v)
                inc_i = jnp.where(prev_gt, ri[j - 1], vi)
                nv = jnp.where(gt, inc_v, rv[j])
                ni = jnp.where(gt, inc_i, ri[j])
            prev_gt = gt
            new_rv.append(nv)
            new_ri.append(ni)
        rv, ri = new_rv, new_ri
    for j in range(TOPK):
        tv_ref[:, j:j + 1] = rv[j]
        ti_ref[:, j:j + 1] = ri[j]

    @pl.when(i == nsteps - 1)
    def _finish():
        m = m_ref[...]
        z = z_ref[...]
        pj = [jnp.exp(rv[j] - m) / z for j in range(TOPK)]
        delta = pj[TOPK - 1] + EPS
        wj = [jnp.maximum(pj[j] - delta, 0.0) for j in range(TOPK)]
        denom = wj[0] + wj[1] + wj[2] + wj[3] + wj[4] + EPS
        inv = 1.0 / denom
        for j in range(TOPK):
            w_ref[:, j:j + 1] = wj[j] * inv
            idx_ref[:, j:j + 1] = ri[j]


def _scatter_kernel(w_ref, idx_ref, out_ref, *, kt):
    i = pl.program_id(0)
    nq = out_ref.shape[0]
    col = jax.lax.broadcasted_iota(jnp.int32, (1, kt), 1) + i * kt
    acc = jnp.zeros((nq, kt), jnp.float32)
    for j in range(TOPK - 1):
        acc = jnp.where(col == idx_ref[:, j:j + 1], w_ref[:, j:j + 1], acc)
    out_ref[...] = acc


def kernel(q, k, W_qs, b_qs, W_ks, b_ks):
    nq, d_model = q.shape
    n_k = k.shape[0]
    d_k = W_qs.shape[0]
    nsteps = pl.cdiv(n_k, KT)
    scale = 1.0 / math.sqrt(d_k)

    wqt = W_qs.T
    wkt = W_ks.T
    bq = b_qs.reshape(1, d_k)
    bk = b_ks.reshape(1, d_k)

    w, idx = pl.pallas_call(
        functools.partial(_stats_kernel, n_k=n_k, kt=KT, nsteps=nsteps,
                          scale=scale),
        grid=(nsteps,),
        in_specs=[
            pl.BlockSpec((nq, d_model), lambda i: (0, 0)),
            pl.BlockSpec((KT, d_model), lambda i: (i, 0)),
            pl.BlockSpec((d_model, d_k), lambda i: (0, 0)),
            pl.BlockSpec((1, d_k), lambda i: (0, 0)),
            pl.BlockSpec((d_model, d_k), lambda i: (0, 0)),
            pl.BlockSpec((1, d_k), lambda i: (0, 0)),
        ],
        out_specs=[
            pl.BlockSpec((nq, TOPK), lambda i: (0, 0)),
            pl.BlockSpec((nq, TOPK), lambda i: (0, 0)),
        ],
        out_shape=[
            jax.ShapeDtypeStruct((nq, TOPK), jnp.float32),
            jax.ShapeDtypeStruct((nq, TOPK), jnp.int32),
        ],
        scratch_shapes=[
            pltpu.VMEM((nq, d_k), jnp.float32),
            pltpu.VMEM((nq, 1), jnp.float32),
            pltpu.VMEM((nq, 1), jnp.float32),
            pltpu.VMEM((nq, TOPK), jnp.float32),
            pltpu.VMEM((nq, TOPK), jnp.int32),
        ],
        compiler_params=pltpu.CompilerParams(
            dimension_semantics=("arbitrary",)),
    )(q, k, wqt, bq, wkt, bk)

    out = pl.pallas_call(
        functools.partial(_scatter_kernel, kt=KT),
        grid=(nsteps,),
        in_specs=[
            pl.BlockSpec((nq, TOPK), lambda i: (0, 0)),
            pl.BlockSpec((nq, TOPK), lambda i: (0, 0)),
        ],
        out_specs=pl.BlockSpec((nq, KT), lambda i: (0, i)),
        out_shape=jax.ShapeDtypeStruct((nq, n_k), jnp.float32),
        compiler_params=pltpu.CompilerParams(
            dimension_semantics=("arbitrary",)),
    )(w, idx)

    return out.reshape(1, nq, n_k), 0.0
